# baseline (speedup 1.0000x reference)
import functools

import jax
import jax.numpy as jnp
from jax import lax
from jax.experimental import pallas as pl
from jax.experimental.pallas import tpu as pltpu
from jax.experimental.pallas import tpu_sc as plsc

_W = 80


def _stage1_kernel(lp_ref, tg_ref, coef_ref, allow_ref, corr_ref, mm_ref):
    T, B, V = lp_ref.shape
    L = tg_ref.shape[1]
    f32 = jnp.float32
    iota_v = jax.lax.broadcasted_iota(jnp.int32, (1, V, V), 1)
    c01 = jax.lax.broadcasted_iota(jnp.int32, (B, 2), 1)
    cneg = jnp.full((B, V - L - 2), -1, jnp.int32)
    kidx = jax.lax.broadcasted_iota(jnp.int32, (1, L), 1)

    text_all = jnp.concatenate([tg_ref[...], c01, cneg], axis=1)
    mcol = (jax.lax.broadcasted_iota(jnp.int32, (1, 1, V), 2) == L + 2)
    mm_ref[...] = (iota_v == text_all.reshape(B, 1, V)).astype(f32) + (
        mcol.astype(f32) * (1.0 / V)
    )

    def stage1(i, corr):
        for j in range(8):
            b = 8 * i + j
            lp_b = lp_ref[:, pl.ds(b, 1), :].reshape(T, V)
            mm = mm_ref[pl.ds(b, 1)].reshape(V, V)
            lpo = jnp.dot(lp_b, mm, preferred_element_type=f32)
            coef_ref[pl.ds(b, 1)] = lpo[:, 0:_W].reshape(1, T, _W)
            tg_b = tg_ref[pl.ds(b, 1), :]
            tprev = jnp.concatenate([tg_b[:, :1], tg_b[:, :-1]], axis=1)
            allow_ref[pl.ds(b, 1)] = ((kidx >= 1) & (tg_b != tprev)).astype(
                f32
            )
            corr = corr + jnp.sum(lpo[:, L + 2 : L + 3])
        return corr

    corr = jax.lax.fori_loop(0, B // 8, stage1, jnp.zeros((), f32))
    corr_ref[...] = jnp.reshape(corr, (1, 1))


def _make_sc_dp(T, B, L):
    CH = 128
    NCH = T // CH
    f32 = jnp.float32
    info = plsc.get_sparse_core_info()
    NC = info.num_cores

    mesh = plsc.VectorSubcoreMesh(core_axis_name="c", subcore_axis_name="s")

    @functools.partial(
        pl.kernel,
        mesh=mesh,
        out_type=jax.ShapeDtypeStruct((B, 16), f32),
        scratch_types=[
            pltpu.VMEM((2, CH, _W), f32),
            pltpu.VMEM((64,), f32),
            pltpu.VMEM((80,), f32),
            pltpu.VMEM((80,), f32),
            pltpu.VMEM((80,), f32),
            pltpu.VMEM((80,), f32),
            pltpu.VMEM((16,), f32),
            pltpu.SemaphoreType.DMA,
            pltpu.SemaphoreType.DMA,
            pltpu.SemaphoreType.DMA,
        ],
    )
    def sc_dp(coef_hbm, allow_hbm, out_hbm, buf, alv, beh, bdeh, bo, bdo,
              res, sem0, sem1, sema):
        b = lax.axis_index("s") * NC + lax.axis_index("c")
        pltpu.async_copy(allow_hbm.at[b], alv, sema).wait()
        allow = [alv[pl.ds(16 * j, 16)] for j in range(4)]
        zero = jnp.zeros((16,), f32)
        bo[pl.ds(0, 16)] = zero
        bdo[pl.ds(0, 16)] = zero
        i0 = jnp.full((16,), 0, jnp.int32)
        i1 = jnp.full((16,), 1, jnp.int32)
        i2 = jnp.full((16,), 2, jnp.int32)

        sems = (sem0, sem1)

        def dma(g, par):
            return pltpu.make_async_copy(
                coef_hbm.at[b, pl.ds(g * CH, CH), :],
                buf.at[par], sems[par],
            )

        dma(0, 0).start()

        def step(bufpar, tloc, st):
            eh, deh, o, do, e0, de0 = st
            lg = [buf[bufpar, tloc, pl.ds(16 * j, 16)] for j in range(4)]
            sv = buf[bufpar, tloc, pl.ds(L, 16)]
            msp = jnp.take(sv, i2)
            l0 = jnp.take(sv, i0)
            l1 = jnp.take(sv, i1)
            po = [jnp.exp(x) for x in lg]
            fo = [po[j] * (lg[j] - msp) for j in range(4)]
            p0 = jnp.exp(l0)
            pab = jnp.exp(l1)
            f0 = p0 * (l0 - msp)
            fab = pab * (l1 - msp)
            se = p0 + pab
            dse = f0 + fab
            beh[pl.ds(0, 16)] = e0
            bdeh[pl.ds(0, 16)] = de0
            for j in range(4):
                beh[pl.ds(8 + 16 * j, 16)] = eh[j]
                bdeh[pl.ds(8 + 16 * j, 16)] = deh[j]
                bo[pl.ds(8 + 16 * j, 16)] = o[j]
                bdo[pl.ds(8 + 16 * j, 16)] = do[j]
            eh_n = [se * eh[j] + p0 * o[j] for j in range(4)]
            deh_n = [se * deh[j] + dse * eh[j] + p0 * do[j] + f0 * o[j]
                     for j in range(4)]
            e0_n = se * e0
            de0_n = se * de0 + dse * e0
            o_n = []
            do_n = []
            for j in range(4):
                esh = beh[pl.ds(7 + 16 * j, 16)]
                desh = bdeh[pl.ds(7 + 16 * j, 16)]
                osh = bo[pl.ds(7 + 16 * j, 16)]
                dosh = bdo[pl.ds(7 + 16 * j, 16)]
                t1 = esh + allow[j] * osh
                dt1 = desh + allow[j] * dosh
                wo = po[j] + pab
                dwo = fo[j] + fab
                o_n.append(wo * o[j] + po[j] * t1)
                do_n.append(wo * do[j] + dwo * o[j] + po[j] * dt1 + fo[j] * t1)
            return eh_n, deh_n, o_n, do_n, e0_n, de0_n

        lane16 = jax.lax.broadcasted_iota(jnp.int32, (16,), 0)

        def norm(st):
            eh, deh, o, do, e0, de0 = st
            s = eh[0] + eh[1] + eh[2] + eh[3] + o[0] + o[1] + o[2] + o[3]
            for sh in (1, 2, 4, 8):
                s = s + jnp.take(s, lane16 ^ sh)
            c = s + e0
            rp = 1048576.0 / c
            return ([x * rp for x in eh], [x * rp for x in deh],
                    [x * rp for x in o], [x * rp for x in do],
                    e0 * rp, de0 * rp)

        def flatten(st):
            eh, deh, o, do, e0, de0 = st
            return (*eh, *deh, *o, *do, e0, de0)

        def unflatten(fl):
            return (list(fl[0:4]), list(fl[4:8]), list(fl[8:12]),
                    list(fl[12:16]), fl[16], fl[17])

        ones = jnp.full((16,), 1.0, f32)
        st = ([zero] * 4, [zero] * 4, [zero] * 4, [zero] * 4, ones, zero)

        def chunk_pair(gp, fl):
            st = unflatten(fl)
            for par in range(2):
                g = 2 * gp + par
                dma(g, par).wait()
                dma(jnp.minimum(g + 1, NCH - 1), (par + 1) % 2).start()

                def run(ii, fl2):
                    st2 = unflatten(fl2)
                    for jj in range(4):
                        st2 = step(par, 4 * ii + jj, st2)
                    return flatten(norm(st2))

                st = unflatten(lax.fori_loop(0, CH // 4, run, flatten(st)))
            return flatten(st)

        fl = lax.fori_loop(0, NCH // 2, chunk_pair, flatten(st))
        dma(NCH - 1, 0).wait()
        eh, deh, o, do, e0, de0 = unflatten(fl)
        res[...] = (deh[3] + do[3]) / (eh[3] + o[3])
        pltpu.sync_copy(res, out_hbm.at[b])

    return sc_dp


@jax.jit
def kernel(log_probs, targets, input_lengths, target_lengths):
    T, B, V = log_probs.shape
    L = targets.shape[1]
    coef, allow, corr = pl.pallas_call(
        _stage1_kernel,
        out_shape=[
            jax.ShapeDtypeStruct((B, T, _W), jnp.float32),
            jax.ShapeDtypeStruct((B, L), jnp.float32),
            jax.ShapeDtypeStruct((1, 1), jnp.float32),
        ],
        scratch_shapes=[pltpu.VMEM((B, V, V), jnp.float32)],
    )(log_probs, targets)
    dvec = _make_sc_dp(T, B, L)(coef, allow)
    total = (jnp.sum(dvec[:, 15]) + corr[0, 0]) / (B * -4.0)
    return total


# speedup vs baseline: 1.0190x; 1.0190x over previous
import functools

import jax
import jax.numpy as jnp
from jax import lax
from jax.experimental import pallas as pl
from jax.experimental.pallas import tpu as pltpu
from jax.experimental.pallas import tpu_sc as plsc

_W = 80


def _stage1_kernel(lp_ref, tg_ref, coef_ref, allow_ref, corr_ref, mm_ref):
    T, B, V = lp_ref.shape
    L = tg_ref.shape[1]
    f32 = jnp.float32
    iota_v = jax.lax.broadcasted_iota(jnp.int32, (1, V, V), 1)
    c01 = jax.lax.broadcasted_iota(jnp.int32, (B, 2), 1)
    cneg = jnp.full((B, V - L - 2), -1, jnp.int32)
    kidx = jax.lax.broadcasted_iota(jnp.int32, (1, L), 1)

    text_all = jnp.concatenate([tg_ref[...], c01, cneg], axis=1)
    mcol = (jax.lax.broadcasted_iota(jnp.int32, (1, 1, V), 2) == L + 2)
    mm_ref[...] = (iota_v == text_all.reshape(B, 1, V)).astype(f32) + (
        mcol.astype(f32) * (1.0 / V)
    )

    def stage1(i, corr):
        for j in range(4):
            b = 4 * i + j
            lp_b = lp_ref[:, pl.ds(b, 1), :].reshape(T, V)
            mm = mm_ref[pl.ds(b, 1)].reshape(V, V)
            lpo = jnp.dot(lp_b, mm, preferred_element_type=f32)
            coef_ref[pl.ds(b, 1)] = lpo[:, 0:_W].reshape(1, T, _W)
            tg_b = tg_ref[pl.ds(b, 1), :]
            tprev = jnp.concatenate([tg_b[:, :1], tg_b[:, :-1]], axis=1)
            allow_ref[pl.ds(b, 1)] = ((kidx >= 1) & (tg_b != tprev)).astype(
                f32
            )
            corr = corr + jnp.sum(lpo[:, L + 2 : L + 3])
        return corr

    corr = jax.lax.fori_loop(0, B // 4, stage1, jnp.zeros((), f32))
    corr_ref[...] = jnp.reshape(corr, (1, 1))


def _make_sc_dp(T, B, L):
    CH = 64
    NCH = T // CH
    f32 = jnp.float32
    info = plsc.get_sparse_core_info()
    NC = info.num_cores

    mesh = plsc.VectorSubcoreMesh(core_axis_name="c", subcore_axis_name="s")

    @functools.partial(
        pl.kernel,
        mesh=mesh,
        out_type=jax.ShapeDtypeStruct((B, 16), f32),
        scratch_types=[
            pltpu.VMEM((2, CH, _W), f32),
            pltpu.VMEM((64,), f32),
            pltpu.VMEM((80,), f32),
            pltpu.VMEM((80,), f32),
            pltpu.VMEM((80,), f32),
            pltpu.VMEM((80,), f32),
            pltpu.VMEM((16,), f32),
            pltpu.SemaphoreType.DMA,
            pltpu.SemaphoreType.DMA,
            pltpu.SemaphoreType.DMA,
        ],
    )
    def sc_dp(coef_hbm, allow_hbm, out_hbm, buf, alv, beh, bdeh, bo, bdo,
              res, sem0, sem1, sema):
        b = lax.axis_index("s") * NC + lax.axis_index("c")
        pltpu.async_copy(allow_hbm.at[b], alv, sema).wait()
        allow = [alv[pl.ds(16 * j, 16)] for j in range(4)]
        zero = jnp.zeros((16,), f32)
        bo[pl.ds(0, 16)] = zero
        bdo[pl.ds(0, 16)] = zero
        i0 = jnp.full((16,), 0, jnp.int32)
        i1 = jnp.full((16,), 1, jnp.int32)
        i2 = jnp.full((16,), 2, jnp.int32)

        sems = (sem0, sem1)

        def dma(g, par):
            return pltpu.make_async_copy(
                coef_hbm.at[b, pl.ds(g * CH, CH), :],
                buf.at[par], sems[par],
            )

        dma(0, 0).start()

        def step(bufpar, tloc, st):
            eh, deh, o, do, e0, de0 = st
            lg = [buf[bufpar, tloc, pl.ds(16 * j, 16)] for j in range(4)]
            sv = buf[bufpar, tloc, pl.ds(L, 16)]
            msp = jnp.take(sv, i2)
            l0 = jnp.take(sv, i0)
            l1 = jnp.take(sv, i1)
            po = [jnp.exp(x) for x in lg]
            fo = [po[j] * (lg[j] - msp) for j in range(4)]
            p0 = jnp.exp(l0)
            pab = jnp.exp(l1)
            f0 = p0 * (l0 - msp)
            fab = pab * (l1 - msp)
            se = p0 + pab
            dse = f0 + fab
            beh[pl.ds(0, 16)] = e0
            bdeh[pl.ds(0, 16)] = de0
            for j in range(4):
                beh[pl.ds(8 + 16 * j, 16)] = eh[j]
                bdeh[pl.ds(8 + 16 * j, 16)] = deh[j]
                bo[pl.ds(8 + 16 * j, 16)] = o[j]
                bdo[pl.ds(8 + 16 * j, 16)] = do[j]
            eh_n = [se * eh[j] + p0 * o[j] for j in range(4)]
            deh_n = [se * deh[j] + dse * eh[j] + p0 * do[j] + f0 * o[j]
                     for j in range(4)]
            e0_n = se * e0
            de0_n = se * de0 + dse * e0
            o_n = []
            do_n = []
            for j in range(4):
                esh = beh[pl.ds(7 + 16 * j, 16)]
                desh = bdeh[pl.ds(7 + 16 * j, 16)]
                osh = bo[pl.ds(7 + 16 * j, 16)]
                dosh = bdo[pl.ds(7 + 16 * j, 16)]
                t1 = esh + allow[j] * osh
                dt1 = desh + allow[j] * dosh
                wo = po[j] + pab
                dwo = fo[j] + fab
                o_n.append(wo * o[j] + po[j] * t1)
                do_n.append(wo * do[j] + dwo * o[j] + po[j] * dt1 + fo[j] * t1)
            return eh_n, deh_n, o_n, do_n, e0_n, de0_n

        lane16 = jax.lax.broadcasted_iota(jnp.int32, (16,), 0)

        def norm(st):
            eh, deh, o, do, e0, de0 = st
            s = eh[0] + eh[1] + eh[2] + eh[3] + o[0] + o[1] + o[2] + o[3]
            for sh in (1, 2, 4, 8):
                s = s + jnp.take(s, lane16 ^ sh)
            c = s + e0
            rp = 1048576.0 / c
            return ([x * rp for x in eh], [x * rp for x in deh],
                    [x * rp for x in o], [x * rp for x in do],
                    e0 * rp, de0 * rp)

        def flatten(st):
            eh, deh, o, do, e0, de0 = st
            return (*eh, *deh, *o, *do, e0, de0)

        def unflatten(fl):
            return (list(fl[0:4]), list(fl[4:8]), list(fl[8:12]),
                    list(fl[12:16]), fl[16], fl[17])

        ones = jnp.full((16,), 1.0, f32)
        st = ([zero] * 4, [zero] * 4, [zero] * 4, [zero] * 4, ones, zero)

        def chunk_pair(gp, fl):
            st = unflatten(fl)
            for par in range(2):
                g = 2 * gp + par
                dma(g, par).wait()
                dma(jnp.minimum(g + 1, NCH - 1), (par + 1) % 2).start()

                def run(ii, fl2):
                    st2 = unflatten(fl2)
                    for jj in range(4):
                        st2 = step(par, 4 * ii + jj, st2)
                    return flatten(norm(st2))

                st = unflatten(lax.fori_loop(0, CH // 4, run, flatten(st)))
            return flatten(st)

        fl = lax.fori_loop(0, NCH // 2, chunk_pair, flatten(st))
        dma(NCH - 1, 0).wait()
        eh, deh, o, do, e0, de0 = unflatten(fl)
        res[...] = (deh[3] + do[3]) / (eh[3] + o[3])
        pltpu.sync_copy(res, out_hbm.at[b])

    return sc_dp


@jax.jit
def kernel(log_probs, targets, input_lengths, target_lengths):
    T, B, V = log_probs.shape
    L = targets.shape[1]
    coef, allow, corr = pl.pallas_call(
        _stage1_kernel,
        out_shape=[
            jax.ShapeDtypeStruct((B, T, _W), jnp.float32),
            jax.ShapeDtypeStruct((B, L), jnp.float32),
            jax.ShapeDtypeStruct((1, 1), jnp.float32),
        ],
        scratch_shapes=[pltpu.VMEM((B, V, V), jnp.float32)],
    )(log_probs, targets)
    dvec = _make_sc_dp(T, B, L)(coef, allow)
    total = (jnp.sum(dvec[:, 15]) + corr[0, 0]) / (B * -4.0)
    return total
